# baseline (speedup 1.0000x reference)
import functools

import jax
import jax.numpy as jnp
import numpy as np
from jax import lax
from jax.experimental import pallas as pl
from jax.experimental.pallas import tpu as pltpu
from jax.experimental.pallas import tpu_sc as plsc

_D = 3
_L = 16
_C = 2
_BASE_RES = 16
_LOG2_HASHMAP = 19
_MAX_PARAMS = 2 ** _LOG2_HASHMAP
_HASH_MASK = _MAX_PARAMS - 1
_OFF = [0]
for _i in range(_L):
    _res = _BASE_RES * (2 ** _i)
    _OFF.append(_OFF[-1] + min(_MAX_PARAMS, (_res + 1) ** _D))
_P1 = np.int32(np.uint32(2654435761))
_P2 = np.int32(np.uint32(805459861))
_B = 131072

_NW = 32
_PPW = _B // _NW
_CH = 2048
_NCHUNK = _PPW // _CH
_NG = _CH // 16


def _hash_body(
    x_hbm, emb0_hbm, emb1_hbm, out_hbm, x_v, f_v, idx_v, rows0_v, rows1_v,
    out_v, sem0, sem1
):
    wid = lax.axis_index("s") * 2 + lax.axis_index("c")
    sems = (sem0, sem1)

    def p1(l, sel):
        res = _BASE_RES * (2 ** l)
        params = _OFF[l + 1] - _OFF[l]
        off = _OFF[l]
        use_hash = (res + 1) ** _D > params
        res_f = jnp.float32(res)

        def body(g, _):
            s = g * 16
            acc_idx = []
            for d in range(3):
                xd = x_v[d, pl.ds(s, 16)]
                pos = ((xd + 1.0) * 0.5) * res_f
                pg = pos.astype(jnp.int32)
                f_v[sel, d, pl.ds(s, 16)] = pos - pg.astype(jnp.float32)
                acc_idx.append(pg)
            i0, i1, i2 = acc_idx
            if use_hash:
                h1 = i1 * _P1
                h1b = h1 + _P1
                h2 = i2 * _P2
                h2b = h2 + _P2
                i0b = i0 + 1
                for corner in range(8):
                    a = i0b if (corner & 1) else i0
                    hh1 = h1b if (corner & 2) else h1
                    hh2 = h2b if (corner & 4) else h2
                    idx = ((a ^ hh1 ^ hh2) & _HASH_MASK) + off
                    idx_v[sel, pl.ds(corner * _CH + s, 16)] = idx
            else:
                s1 = np.int32(res + 1)
                s2 = np.int32((res + 1) * (res + 1))
                h1 = i1 * s1
                h1b = h1 + s1
                h2 = i2 * s2 + np.int32(off)
                h2b = h2 + s2
                i0b = i0 + 1
                for corner in range(8):
                    a = i0b if (corner & 1) else i0
                    hh1 = h1b if (corner & 2) else h1
                    hh2 = h2b if (corner & 4) else h2
                    idx_v[sel, pl.ds(corner * _CH + s, 16)] = a + hh1 + hh2
            return 0

        lax.fori_loop(0, _NG, body, 0, unroll=4)

    def fire(sel):
        c0 = pltpu.async_copy(emb0_hbm.at[idx_v.at[sel]], rows0_v.at[sel],
                              sems[sel])
        c1 = pltpu.async_copy(emb1_hbm.at[idx_v.at[sel]], rows1_v.at[sel],
                              sems[sel])
        return (c0, c1)

    def p2(l, sel, cbase):
        def body(g, _):
            s = g * 16
            f0 = f_v[sel, 0, pl.ds(s, 16)]
            f1 = f_v[sel, 1, pl.ds(s, 16)]
            f2 = f_v[sel, 2, pl.ds(s, 16)]
            g0 = 1.0 - f0
            g1 = 1.0 - f1
            g2 = 1.0 - f2
            w01 = (g0 * g1, f0 * g1, g0 * f1, f0 * f1)
            acc0 = None
            acc1 = None
            for corner in range(8):
                w2 = f2 if (corner & 4) else g2
                w = w01[corner & 3] * w2
                r0 = rows0_v[sel, pl.ds(corner * _CH + s, 16)]
                r1 = rows1_v[sel, pl.ds(corner * _CH + s, 16)]
                if acc0 is None:
                    acc0 = w * r0
                    acc1 = w * r1
                else:
                    acc0 = acc0 + w * r0
                    acc1 = acc1 + w * r1
            out_v[0, pl.ds(s, 16)] = acc0
            out_v[1, pl.ds(s, 16)] = acc1
            return 0

        lax.fori_loop(0, _NG, body, 0, unroll=4)
        pltpu.sync_copy(out_v, out_hbm.at[l, :, pl.ds(cbase, _CH)])

    def chunk_body(ck, _):
        cbase = wid * _PPW + ck * _CH
        pltpu.sync_copy(x_hbm.at[:, pl.ds(cbase, _CH)], x_v)

        p1(0, 0)
        inflight = fire(0)
        for l in range(1, _L):
            sel = l % 2
            prev = 1 - sel
            p1(l, sel)
            nxt = fire(sel)
            inflight[0].wait()
            inflight[1].wait()
            p2(l - 1, prev, cbase)
            inflight = nxt
        inflight[0].wait()
        inflight[1].wait()
        p2(_L - 1, (_L - 1) % 2, cbase)
        return 0

    lax.fori_loop(0, _NCHUNK, chunk_body, 0)


@functools.cache
def _build_encode_sc():
    mesh = plsc.VectorSubcoreMesh(core_axis_name="c", subcore_axis_name="s")
    return functools.partial(
        pl.kernel,
        out_type=jax.ShapeDtypeStruct((_L, _C, _B), jnp.float32),
        mesh=mesh,
        compiler_params=pltpu.CompilerParams(
            needs_layout_passes=False, use_tc_tiling_on_sc=False
        ),
        scratch_types=[
            pltpu.VMEM((3, _CH), jnp.float32),
            pltpu.VMEM((2, 3, _CH), jnp.float32),
            pltpu.VMEM((2, 8 * _CH), jnp.int32),
            pltpu.VMEM((2, 8 * _CH), jnp.float32),
            pltpu.VMEM((2, 8 * _CH), jnp.float32),
            pltpu.VMEM((2, _CH), jnp.float32),
            pltpu.SemaphoreType.DMA,
            pltpu.SemaphoreType.DMA,
        ],
    )(_hash_body)


@jax.jit
def kernel(inputs, embeddings):
    x_t = inputs.T
    emb0 = embeddings[:, 0]
    emb1 = embeddings[:, 1]
    out = _build_encode_sc()(x_t, emb0, emb1)
    return out.transpose(2, 0, 1).reshape(_B, _L * _C)


# speedup vs baseline: 1.8516x; 1.8516x over previous
import functools

import jax
import jax.numpy as jnp
import numpy as np
from jax import lax
from jax.experimental import pallas as pl
from jax.experimental.pallas import tpu as pltpu
from jax.experimental.pallas import tpu_sc as plsc

_D = 3
_L = 16
_C = 2
_BASE_RES = 16
_LOG2_HASHMAP = 19
_MAX_PARAMS = 2 ** _LOG2_HASHMAP
_HASH_MASK = _MAX_PARAMS - 1
_OFF = [0]
for _i in range(_L):
    _res = _BASE_RES * (2 ** _i)
    _OFF.append(_OFF[-1] + min(_MAX_PARAMS, (_res + 1) ** _D))
_P1 = np.int32(np.uint32(2654435761))
_P2 = np.int32(np.uint32(805459861))
_B = 131072

_NW = 32
_PPW = _B // _NW
_CH = 1024
_NCHUNK = _PPW // _CH
_NG = _CH // 16


def _hash_body(
    x_hbm, embp_hbm, out_hbm, x_v, f_v, idx_v, rows_v, out_v, sem0, sem1
):
    wid = lax.axis_index("s") * 2 + lax.axis_index("c")
    sems = (sem0, sem1)

    def p1(l, sel):
        res = _BASE_RES * (2 ** l)
        params = _OFF[l + 1] - _OFF[l]
        off = _OFF[l]
        use_hash = (res + 1) ** _D > params
        res_f = jnp.float32(res)

        def body(g, _):
            s = g * 16
            acc_idx = []
            for d in range(3):
                xd = x_v[d, pl.ds(s, 16)]
                pos = ((xd + 1.0) * 0.5) * res_f
                pg = pos.astype(jnp.int32)
                f_v[sel, d, pl.ds(s, 16)] = pos - pg.astype(jnp.float32)
                acc_idx.append(pg)
            i0, i1, i2 = acc_idx
            if use_hash:
                h1 = i1 * _P1
                h1b = h1 + _P1
                h2 = i2 * _P2
                h2b = h2 + _P2
                i0b = i0 + 1
                for corner in range(8):
                    a = i0b if (corner & 1) else i0
                    hh1 = h1b if (corner & 2) else h1
                    hh2 = h2b if (corner & 4) else h2
                    idx = ((a ^ hh1 ^ hh2) & _HASH_MASK) + off
                    idx_v[sel, pl.ds(corner * _CH + s, 16)] = idx
            else:
                s1 = np.int32(res + 1)
                s2 = np.int32((res + 1) * (res + 1))
                h1 = i1 * s1
                h1b = h1 + s1
                h2 = i2 * s2 + np.int32(off)
                h2b = h2 + s2
                i0b = i0 + 1
                for corner in range(8):
                    a = i0b if (corner & 1) else i0
                    hh1 = h1b if (corner & 2) else h1
                    hh2 = h2b if (corner & 4) else h2
                    idx_v[sel, pl.ds(corner * _CH + s, 16)] = a + hh1 + hh2
            return 0

        lax.fori_loop(0, _NG, body, 0, unroll=4)

    def fire(sel):
        return pltpu.async_copy(
            embp_hbm.at[idx_v.at[sel]], rows_v.at[sel], sems[sel]
        )

    def p2(l, sel, cbase):
        def body(g, _):
            s = g * 16
            f0 = f_v[sel, 0, pl.ds(s, 16)]
            f1 = f_v[sel, 1, pl.ds(s, 16)]
            f2 = f_v[sel, 2, pl.ds(s, 16)]
            g0 = 1.0 - f0
            g1 = 1.0 - f1
            g2 = 1.0 - f2
            w01 = (g0 * g1, f0 * g1, g0 * f1, f0 * f1)
            acc0 = None
            acc1 = None
            for corner in range(8):
                w2 = f2 if (corner & 4) else g2
                w = w01[corner & 3] * w2
                rp = rows_v[sel, pl.ds(corner * _CH + s, 16)]
                r0, r1 = plsc.unpack(
                    plsc.bitcast(rp, jnp.bfloat16),
                    format=plsc.PackFormat.INTERLEAVED,
                )
                if acc0 is None:
                    acc0 = w * r0
                    acc1 = w * r1
                else:
                    acc0 = acc0 + w * r0
                    acc1 = acc1 + w * r1
            out_v[0, pl.ds(s, 16)] = acc0
            out_v[1, pl.ds(s, 16)] = acc1
            return 0

        lax.fori_loop(0, _NG, body, 0, unroll=4)
        pltpu.sync_copy(out_v, out_hbm.at[l, :, pl.ds(cbase, _CH)])

    def chunk_body(ck, _):
        cbase = wid * _PPW + ck * _CH
        pltpu.sync_copy(x_hbm.at[:, pl.ds(cbase, _CH)], x_v)

        p1(0, 0)
        inflight = fire(0)
        for l in range(1, _L):
            sel = l % 2
            prev = 1 - sel
            p1(l, sel)
            nxt = fire(sel)
            inflight.wait()
            p2(l - 1, prev, cbase)
            inflight = nxt
        inflight.wait()
        p2(_L - 1, (_L - 1) % 2, cbase)
        return 0

    lax.fori_loop(0, _NCHUNK, chunk_body, 0)


@functools.cache
def _build_encode_sc():
    mesh = plsc.VectorSubcoreMesh(core_axis_name="c", subcore_axis_name="s")
    return functools.partial(
        pl.kernel,
        out_type=jax.ShapeDtypeStruct((_L, _C, _B), jnp.float32),
        mesh=mesh,
        compiler_params=pltpu.CompilerParams(
            needs_layout_passes=False, use_tc_tiling_on_sc=False
        ),
        scratch_types=[
            pltpu.VMEM((3, _CH), jnp.float32),
            pltpu.VMEM((2, 3, _CH), jnp.float32),
            pltpu.VMEM((2, 8 * _CH), jnp.int32),
            pltpu.VMEM((2, 8 * _CH), jnp.int32),
            pltpu.VMEM((2, _CH), jnp.float32),
            pltpu.SemaphoreType.DMA,
            pltpu.SemaphoreType.DMA,
        ],
    )(_hash_body)


@jax.jit
def kernel(inputs, embeddings):
    x_t = inputs.T
    u0 = lax.bitcast_convert_type(
        embeddings[:, 0].astype(jnp.bfloat16), jnp.uint16
    ).astype(jnp.uint32)
    u1 = lax.bitcast_convert_type(
        embeddings[:, 1].astype(jnp.bfloat16), jnp.uint16
    ).astype(jnp.uint32)
    embp = lax.bitcast_convert_type(u0 | (u1 << 16), jnp.int32)
    out = _build_encode_sc()(x_t, embp)
    return out.transpose(2, 0, 1).reshape(_B, _L * _C)
